# baseline (speedup 1.0000x reference)
import functools

import jax
import jax.numpy as jnp
from jax.experimental import pallas as pl
from jax.experimental.pallas import tpu as pltpu


def _loss_kernel(total_cells, nsteps, x_ref, y_ref, out_ref, acc_ref):
    step = pl.program_id(0)

    @pl.when(step == 0)
    def _init():
        acc_ref[0] = 0.0
        acc_ref[1] = 0.0
        acc_ref[2] = 0.0
        acc_ref[3] = 0.0

    cx = x_ref[0]
    cy = y_ref[0]
    mask = (cy > 0.5).astype(jnp.float32)

    d = x_ref[1] - y_ref[1]
    sq = d * d
    d = x_ref[2] - y_ref[2]
    sq = sq + d * d
    d = x_ref[3] - y_ref[3]
    sq = sq + d * d
    d = x_ref[4] - y_ref[4]
    sq = sq + d * d

    logp = jnp.maximum(jnp.log(cx), -100.0)
    log1mp = jnp.maximum(jnp.log(1.0 - cx), -100.0)

    acc_ref[0] += jnp.sum(mask)
    acc_ref[1] += jnp.sum(mask * sq)
    acc_ref[2] += jnp.sum(mask * (cy * logp + (1.0 - cy) * log1mp))
    acc_ref[3] += jnp.sum((mask - 1.0) * log1mp)

    @pl.when(step == nsteps - 1)
    def _finalize():
        f = acc_ref[0]
        bg_num = total_cells - f
        loss = (1.0 + 1.0 / f) * ((0.25 * acc_ref[1] - acc_ref[2]) / f)
        out_ref[0, 0] = loss + acc_ref[3] / bg_num


@jax.jit
def kernel(x, y):
    B, N, C = x.shape
    xt = x.transpose(2, 0, 1)
    yt = y.transpose(2, 0, 1)

    nsteps = 1
    rb = B // nsteps

    out = pl.pallas_call(
        functools.partial(_loss_kernel, float(B * N), nsteps),
        grid=(nsteps,),
        out_shape=jax.ShapeDtypeStruct((1, 1), jnp.float32),
        in_specs=[
            pl.BlockSpec((C, rb, N), lambda i: (0, i, 0)),
            pl.BlockSpec((C, rb, N), lambda i: (0, i, 0)),
        ],
        out_specs=pl.BlockSpec(memory_space=pltpu.SMEM),
        scratch_shapes=[pltpu.SMEM((4,), jnp.float32)],
    )(xt, yt)
    return out[0, 0]


# speedup vs baseline: 1.1457x; 1.1457x over previous
import functools

import jax
import jax.numpy as jnp
from jax.experimental import pallas as pl
from jax.experimental.pallas import tpu as pltpu


def _loss_kernel(total_cells, nsteps, x_ref, y_ref, out_ref, acc_ref):
    step = pl.program_id(0)

    @pl.when(step == 0)
    def _init():
        acc_ref[0] = 0.0
        acc_ref[1] = 0.0
        acc_ref[2] = 0.0
        acc_ref[3] = 0.0

    cx = x_ref[0]
    cy = y_ref[0]
    mask = (cy > 0.5).astype(jnp.float32)

    d = x_ref[1] - y_ref[1]
    sq = d * d
    d = x_ref[2] - y_ref[2]
    sq = sq + d * d
    d = x_ref[3] - y_ref[3]
    sq = sq + d * d
    d = x_ref[4] - y_ref[4]
    sq = sq + d * d

    logp = jnp.maximum(jnp.log(cx), -100.0)
    log1mp = jnp.maximum(jnp.log(1.0 - cx), -100.0)

    acc_ref[0] += jnp.sum(mask)
    acc_ref[1] += jnp.sum(mask * sq)
    acc_ref[2] += jnp.sum(mask * (cy * logp + (1.0 - cy) * log1mp))
    acc_ref[3] += jnp.sum((mask - 1.0) * log1mp)

    @pl.when(step == nsteps - 1)
    def _finalize():
        f = acc_ref[0]
        bg_num = total_cells - f
        loss = (1.0 + 1.0 / f) * ((0.25 * acc_ref[1] - acc_ref[2]) / f)
        out_ref[0, 0] = loss + acc_ref[3] / bg_num


@jax.jit
def kernel(x, y):
    B, N, C = x.shape
    xt = x.transpose(2, 0, 1)
    yt = y.transpose(2, 0, 1)

    nsteps = 2
    rb = B // nsteps

    out = pl.pallas_call(
        functools.partial(_loss_kernel, float(B * N), nsteps),
        grid=(nsteps,),
        out_shape=jax.ShapeDtypeStruct((1, 1), jnp.float32),
        in_specs=[
            pl.BlockSpec((C, rb, N), lambda i: (0, i, 0)),
            pl.BlockSpec((C, rb, N), lambda i: (0, i, 0)),
        ],
        out_specs=pl.BlockSpec(memory_space=pltpu.SMEM),
        scratch_shapes=[pltpu.SMEM((4,), jnp.float32)],
    )(xt, yt)
    return out[0, 0]


# speedup vs baseline: 1.1478x; 1.0018x over previous
import functools

import jax
import jax.numpy as jnp
from jax.experimental import pallas as pl
from jax.experimental.pallas import tpu as pltpu

_CHUNKS = ((0, 24), (24, 32), (56, 8))


def _partial_sums(xb, yb, acc):
    face, mse, bpos, bbg = acc
    cx = xb[0]
    cy = yb[0]
    mask = (cy > 0.5).astype(jnp.float32)

    d = xb[1] - yb[1]
    sq = d * d
    d = xb[2] - yb[2]
    sq = sq + d * d
    d = xb[3] - yb[3]
    sq = sq + d * d
    d = xb[4] - yb[4]
    sq = sq + d * d

    logp = jnp.maximum(jnp.log(cx), -100.0)
    log1mp = jnp.maximum(jnp.log(1.0 - cx), -100.0)

    face = face + jnp.sum(mask)
    mse = mse + jnp.sum(mask * sq)
    bpos = bpos + jnp.sum(mask * (cy * logp + (1.0 - cy) * log1mp))
    bbg = bbg + jnp.sum((mask - 1.0) * log1mp)
    return face, mse, bpos, bbg


def _loss_kernel(total_cells, x_hbm, y_hbm, out_ref,
                 xa, ya, xb, yb, xc, yc, *sems):
    bufs = ((xa, ya), (xb, yb), (xc, yc))
    copies = []
    for i, (r0, nr) in enumerate(_CHUNKS):
        copies.append(pltpu.make_async_copy(
            x_hbm.at[:, pl.ds(r0, nr), :], bufs[i][0], sems[2 * i]))
        copies.append(pltpu.make_async_copy(
            y_hbm.at[:, pl.ds(r0, nr), :], bufs[i][1], sems[2 * i + 1]))
    for cp in copies:
        cp.start()

    acc = (0.0, 0.0, 0.0, 0.0)
    for i in range(len(_CHUNKS)):
        copies[2 * i].wait()
        copies[2 * i + 1].wait()
        acc = _partial_sums(bufs[i][0], bufs[i][1], acc)

    face, mse, bpos, bbg = acc
    bg_num = total_cells - face
    loss = (1.0 + 1.0 / face) * ((0.25 * mse - bpos) / face)
    out_ref[0, 0] = loss + bbg / bg_num


@jax.jit
def kernel(x, y):
    B, N, C = x.shape
    xt = x.transpose(2, 0, 1)
    yt = y.transpose(2, 0, 1)

    scratch = []
    for _, nr in _CHUNKS:
        scratch.append(pltpu.VMEM((C, nr, N), jnp.float32))
        scratch.append(pltpu.VMEM((C, nr, N), jnp.float32))
    scratch.extend([pltpu.SemaphoreType.DMA] * (2 * len(_CHUNKS)))

    out = pl.pallas_call(
        functools.partial(_loss_kernel, float(B * N)),
        out_shape=jax.ShapeDtypeStruct((1, 1), jnp.float32),
        in_specs=[
            pl.BlockSpec(memory_space=pl.ANY),
            pl.BlockSpec(memory_space=pl.ANY),
        ],
        out_specs=pl.BlockSpec(memory_space=pltpu.SMEM),
        scratch_shapes=scratch,
    )(xt, yt)
    return out[0, 0]
